# baseline (device time: 174511 ns/iter reference)
import jax
import jax.numpy as jnp
from jax import lax
from jax.experimental import pallas as pl
from jax.experimental.pallas import tpu as pltpu

N_Z = 4


def kernel(x, W):
    t, d = x.shape
    _, v = W.shape
    v_global = N_Z * v

    def body(x_ref, w_ref, out_ref, send_sems, recv_sems):
        my_x = lax.axis_index("x")
        my_y = lax.axis_index("y")
        my_z = lax.axis_index("z")
        left_z = (my_z - 1) % N_Z
        right_z = (my_z + 1) % N_Z

        out_ref[:, pl.ds(my_z * v, v)] = jnp.dot(
            x_ref[:, :], w_ref[:, :], preferred_element_type=jnp.float32
        )

        barrier_sem = pltpu.get_barrier_semaphore()
        for nz in (left_z, right_z):
            pl.semaphore_signal(
                barrier_sem,
                inc=1,
                device_id=(my_x, my_y, nz),
                device_id_type=pl.DeviceIdType.MESH,
            )
        pl.semaphore_wait(barrier_sem, 2)

        rdmas = []
        part_m = []
        part_s = []

        def chunk_partials(origin):
            c = out_ref[:, pl.ds(origin * v, v)]
            m_c = jnp.max(c, axis=1, keepdims=True)
            s_c = jnp.sum(jnp.exp(c - m_c), axis=1, keepdims=True)
            part_m.append(m_c)
            part_s.append(s_c)

        for h in range(N_Z - 1):
            origin = (my_z - h) % N_Z
            rdma = pltpu.make_async_remote_copy(
                src_ref=out_ref.at[:, pl.ds(origin * v, v)],
                dst_ref=out_ref.at[:, pl.ds(origin * v, v)],
                send_sem=send_sems.at[h],
                recv_sem=recv_sems.at[h],
                device_id=(my_x, my_y, right_z),
                device_id_type=pl.DeviceIdType.MESH,
            )
            rdma.start()
            rdmas.append(rdma)
            chunk_partials(origin)
            rdma.wait_recv()
        chunk_partials((my_z + 1) % N_Z)

        m = part_m[0]
        for m_c in part_m[1:]:
            m = jnp.maximum(m, m_c)
        s = part_s[0] * jnp.exp(part_m[0] - m)
        for m_c, s_c in zip(part_m[1:], part_s[1:]):
            s = s + s_c * jnp.exp(m_c - m)
        inv_s = 1.0 / s
        for k in range(N_Z):
            blk = out_ref[:, k * v:(k + 1) * v]
            out_ref[:, k * v:(k + 1) * v] = jnp.exp(blk - m) * inv_s

        for rdma in rdmas:
            rdma.wait_send()

    return pl.pallas_call(
        body,
        out_shape=jax.ShapeDtypeStruct((t, v_global), jnp.float32),
        in_specs=[
            pl.BlockSpec(memory_space=pltpu.VMEM),
            pl.BlockSpec(memory_space=pltpu.VMEM),
        ],
        out_specs=pl.BlockSpec(memory_space=pltpu.VMEM),
        scratch_shapes=[
            pltpu.SemaphoreType.DMA((N_Z - 1,)),
            pltpu.SemaphoreType.DMA((N_Z - 1,)),
        ],
        compiler_params=pltpu.CompilerParams(collective_id=0),
    )(x, W)
